# baseline (speedup 1.0000x reference)
import functools

import jax
import jax.numpy as jnp
from jax import lax
from jax.experimental import pallas as pl
from jax.experimental.pallas import tpu as pltpu
from jax.experimental.pallas import tpu_sc as plsc

VOCAB_SIZE = 32000
D_MODEL = 2048
N_TOKENS = 16384
EPS = 1e-05

_NC = 2
_NS = 16
_NW = _NC * _NS
_B_PER_W = N_TOKENS // _NW
_CHUNK = 16
_NCHUNK = _B_PER_W // _CHUNK


_NBUF = 3


def _sc_gather_body(idx_hbm, table_hbm, out_hbm, idx_v,
                    buf0, buf1, buf2, gsem0, gsem1, gsem2,
                    osem0, osem1, osem2):
    wid = lax.axis_index("s") * _NC + lax.axis_index("c")
    base = wid * _B_PER_W
    pltpu.sync_copy(idx_hbm.at[pl.ds(base, _B_PER_W)], idx_v)

    bufs = (buf0, buf1, buf2)
    gsems = (gsem0, gsem1, gsem2)
    osems = (osem0, osem1, osem2)

    def start_g(c, b):
        pltpu.async_copy(
            table_hbm.at[idx_v.at[pl.ds(c * _CHUNK, _CHUNK)]], bufs[b],
            gsems[b])

    def start_o(c, b):
        pltpu.async_copy(bufs[b],
                         out_hbm.at[pl.ds(base + c * _CHUNK, _CHUNK)],
                         osems[b])

    def wait(b, sem):
        pltpu.make_async_copy(table_hbm.at[pl.ds(0, _CHUNK)], bufs[b],
                              sem).wait()

    for b in range(_NBUF):
        start_g(b, b)

    @pl.loop(0, _NCHUNK - 2, step=_NBUF)
    def _(c):
        for b in range(_NBUF):
            wait(b, gsems[b])
            start_o(c + b, b)
        for b in range(_NBUF):
            wait(b, osems[b])

            @pl.when(c + b + _NBUF < _NCHUNK)
            def _():
                start_g(c + b + _NBUF, b)

    wait(0, gsems[0])
    start_o(_NCHUNK - 2, 0)
    wait(1, gsems[1])
    start_o(_NCHUNK - 1, 1)
    wait(0, osems[0])
    wait(1, osems[1])


def _make_sc_gather(interpret=False):
    return functools.partial(
        pl.kernel,
        out_type=jax.ShapeDtypeStruct((N_TOKENS, D_MODEL), jnp.float32),
        mesh=plsc.VectorSubcoreMesh(
            core_axis_name="c", subcore_axis_name="s",
            num_cores=_NC, num_subcores=_NS),
        scratch_types=(
            [pltpu.VMEM((_B_PER_W,), jnp.int32)]
            + [pltpu.VMEM((_CHUNK, D_MODEL), jnp.float32)] * _NBUF
            + [pltpu.SemaphoreType.DMA] * (2 * _NBUF)
        ),
        cost_estimate=pl.CostEstimate(
            flops=0, transcendentals=0,
            bytes_accessed=2 * N_TOKENS * D_MODEL * 4 + N_TOKENS * 4),
        interpret=interpret,
    )(_sc_gather_body)


_SC_GATHER_CACHE = {}


def _sc_gather(input_ids, embed_table):
    if "k" not in _SC_GATHER_CACHE:
        _SC_GATHER_CACHE["k"] = _make_sc_gather()
    return _SC_GATHER_CACHE["k"](input_ids, embed_table)


_ROW_BLK = 512


def _rms_body(x_ref, w_ref, o_ref):
    x = x_ref[...]
    var = jnp.mean(x * x, axis=-1, keepdims=True)
    o_ref[...] = x * lax.rsqrt(var + EPS) * w_ref[...]


def _rms_norm(hidden_states, norm_weight):
    return pl.pallas_call(
        _rms_body,
        grid=(N_TOKENS // _ROW_BLK,),
        in_specs=[
            pl.BlockSpec((_ROW_BLK, D_MODEL), lambda i: (i, 0)),
            pl.BlockSpec((1, D_MODEL), lambda i: (0, 0)),
        ],
        out_specs=pl.BlockSpec((_ROW_BLK, D_MODEL), lambda i: (i, 0)),
        out_shape=jax.ShapeDtypeStruct((N_TOKENS, D_MODEL), jnp.float32),
    )(hidden_states, norm_weight.reshape(1, D_MODEL))


def kernel(input_ids, positions, hidden_states, embed_table, norm_weight):
    input_embeds = _sc_gather(input_ids, embed_table)
    normed = _rms_norm(hidden_states, norm_weight)
    return (normed, hidden_states, input_embeds)


# speedup vs baseline: 1.0061x; 1.0061x over previous
import functools

import jax
import jax.numpy as jnp
from jax import lax
from jax.experimental import pallas as pl
from jax.experimental.pallas import tpu as pltpu
from jax.experimental.pallas import tpu_sc as plsc

VOCAB_SIZE = 32000
D_MODEL = 2048
N_TOKENS = 16384
EPS = 1e-05

_NC = 2
_NS = 16
_NW = _NC * _NS
_B_PER_W = N_TOKENS // _NW
_CHUNK = 16
_NCHUNK = _B_PER_W // _CHUNK


_NBUF = 3


def _sc_gather_body(idx_hbm, table_hbm, out_hbm, idx_v,
                    buf0, buf1, buf2, gsem0, gsem1, gsem2,
                    osem0, osem1, osem2):
    wid = lax.axis_index("s") * _NC + lax.axis_index("c")
    base = wid * _B_PER_W
    pltpu.sync_copy(idx_hbm.at[pl.ds(base, _B_PER_W)], idx_v)

    bufs = (buf0, buf1, buf2)
    gsems = (gsem0, gsem1, gsem2)
    osems = (osem0, osem1, osem2)

    def start_g(c, b):
        pltpu.async_copy(
            table_hbm.at[idx_v.at[pl.ds(c * _CHUNK, _CHUNK)]], bufs[b],
            gsems[b])

    def start_o(c, b):
        pltpu.async_copy(bufs[b],
                         out_hbm.at[pl.ds(base + c * _CHUNK, _CHUNK)],
                         osems[b])

    def wait(b, sem):
        pltpu.make_async_copy(table_hbm.at[pl.ds(0, _CHUNK)], bufs[b],
                              sem).wait()

    for b in range(_NBUF):
        start_g(b, b)

    @pl.loop(0, _NCHUNK - 2, step=_NBUF)
    def _(c):
        for b in range(_NBUF):
            wait(b, gsems[b])
            start_o(c + b, b)
        for b in range(_NBUF):
            wait(b, osems[b])

            @pl.when(c + b + _NBUF < _NCHUNK)
            def _():
                start_g(c + b + _NBUF, b)

    wait(0, gsems[0])
    start_o(_NCHUNK - 2, 0)
    wait(1, gsems[1])
    start_o(_NCHUNK - 1, 1)
    wait(0, osems[0])
    wait(1, osems[1])


def _make_sc_gather(interpret=False):
    return functools.partial(
        pl.kernel,
        out_type=jax.ShapeDtypeStruct((N_TOKENS, D_MODEL), jnp.float32),
        mesh=plsc.VectorSubcoreMesh(
            core_axis_name="c", subcore_axis_name="s",
            num_cores=_NC, num_subcores=_NS),
        scratch_types=(
            [pltpu.VMEM((_B_PER_W,), jnp.int32)]
            + [pltpu.VMEM((_CHUNK, D_MODEL), jnp.float32)] * _NBUF
            + [pltpu.SemaphoreType.DMA] * (2 * _NBUF)
        ),
        cost_estimate=pl.CostEstimate(
            flops=0, transcendentals=0,
            bytes_accessed=2 * N_TOKENS * D_MODEL * 4 + N_TOKENS * 4),
        interpret=interpret,
    )(_sc_gather_body)


_SC_GATHER_CACHE = {}


def _sc_gather(input_ids, embed_table):
    if "k" not in _SC_GATHER_CACHE:
        _SC_GATHER_CACHE["k"] = _make_sc_gather()
    return _SC_GATHER_CACHE["k"](input_ids, embed_table)


_ROW_BLK = 1024


def _rms_body(x_ref, w_ref, o_ref):
    x = x_ref[...]
    var = jnp.mean(x * x, axis=-1, keepdims=True)
    o_ref[...] = x * lax.rsqrt(var + EPS) * w_ref[...]


def _rms_norm(hidden_states, norm_weight):
    return pl.pallas_call(
        _rms_body,
        grid=(N_TOKENS // _ROW_BLK,),
        in_specs=[
            pl.BlockSpec((_ROW_BLK, D_MODEL), lambda i: (i, 0)),
            pl.BlockSpec((1, D_MODEL), lambda i: (0, 0)),
        ],
        out_specs=pl.BlockSpec((_ROW_BLK, D_MODEL), lambda i: (i, 0)),
        out_shape=jax.ShapeDtypeStruct((N_TOKENS, D_MODEL), jnp.float32),
    )(hidden_states, norm_weight.reshape(1, D_MODEL))


def kernel(input_ids, positions, hidden_states, embed_table, norm_weight):
    input_embeds = _sc_gather(input_ids, embed_table)
    normed = _rms_norm(hidden_states, norm_weight)
    return (normed, hidden_states, input_embeds)


# speedup vs baseline: 1.0148x; 1.0087x over previous
import functools

import jax
import jax.numpy as jnp
from jax import lax
from jax.experimental import pallas as pl
from jax.experimental.pallas import tpu as pltpu
from jax.experimental.pallas import tpu_sc as plsc

VOCAB_SIZE = 32000
D_MODEL = 2048
N_TOKENS = 16384
EPS = 1e-05

_NC = 2
_NS = 16
_NW = _NC * _NS
_B_PER_W = N_TOKENS // _NW
_CHUNK = 24
_NMAIN = _B_PER_W // _CHUNK
_TAIL = _B_PER_W - _NMAIN * _CHUNK


def _sc_gather_body(idx_hbm, table_hbm, out_hbm, idx_v, buf0, buf1,
                    gsem0, gsem1):
    wid = lax.axis_index("s") * _NC + lax.axis_index("c")
    base = wid * _B_PER_W
    pltpu.sync_copy(idx_hbm.at[pl.ds(base, _B_PER_W)], idx_v)

    bufs = (buf0, buf1)
    gsems = (gsem0, gsem1)

    def start_g(c, b):
        pltpu.async_copy(
            table_hbm.at[idx_v.at[pl.ds(c * _CHUNK, _CHUNK)]], bufs[b],
            gsems[b])

    def start_tail(b):
        pltpu.async_copy(
            table_hbm.at[idx_v.at[pl.ds(_NMAIN * _CHUNK, _TAIL)]],
            bufs[b].at[pl.ds(0, _TAIL)], gsems[b])

    def wait_g(b):
        pltpu.make_async_copy(table_hbm.at[pl.ds(0, _CHUNK)], bufs[b],
                              gsems[b]).wait()

    def wait_tail(b):
        pltpu.make_async_copy(table_hbm.at[pl.ds(0, _TAIL)],
                              bufs[b].at[pl.ds(0, _TAIL)], gsems[b]).wait()

    def drain(c, b):
        pltpu.sync_copy(bufs[b], out_hbm.at[pl.ds(base + c * _CHUNK, _CHUNK)])

    start_g(0, 0)

    @pl.loop(0, _NMAIN - 1, step=2)
    def _(c):
        wait_g(0)
        start_g(c + 1, 1)
        drain(c, 0)
        wait_g(1)
        start_g(c + 2, 0)
        drain(c + 1, 1)

    wait_g(0)
    start_tail(1)
    drain(_NMAIN - 1, 0)
    wait_tail(1)
    pltpu.sync_copy(bufs[1].at[pl.ds(0, _TAIL)],
                    out_hbm.at[pl.ds(base + _NMAIN * _CHUNK, _TAIL)])


def _make_sc_gather(interpret=False):
    return functools.partial(
        pl.kernel,
        out_type=jax.ShapeDtypeStruct((N_TOKENS, D_MODEL), jnp.float32),
        mesh=plsc.VectorSubcoreMesh(
            core_axis_name="c", subcore_axis_name="s",
            num_cores=_NC, num_subcores=_NS),
        scratch_types=(
            [pltpu.VMEM((_B_PER_W,), jnp.int32)]
            + [pltpu.VMEM((_CHUNK, D_MODEL), jnp.float32)] * 2
            + [pltpu.SemaphoreType.DMA] * 2
        ),
        cost_estimate=pl.CostEstimate(
            flops=0, transcendentals=0,
            bytes_accessed=2 * N_TOKENS * D_MODEL * 4 + N_TOKENS * 4),
        interpret=interpret,
    )(_sc_gather_body)


_SC_GATHER_CACHE = {}


def _sc_gather(input_ids, embed_table):
    if "k" not in _SC_GATHER_CACHE:
        _SC_GATHER_CACHE["k"] = _make_sc_gather()
    return _SC_GATHER_CACHE["k"](input_ids, embed_table)


_ROW_BLK = 1024


def _rms_body(x_ref, w_ref, o_ref):
    x = x_ref[...]
    var = jnp.mean(x * x, axis=-1, keepdims=True)
    o_ref[...] = x * lax.rsqrt(var + EPS) * w_ref[...]


def _rms_norm(hidden_states, norm_weight):
    return pl.pallas_call(
        _rms_body,
        grid=(N_TOKENS // _ROW_BLK,),
        in_specs=[
            pl.BlockSpec((_ROW_BLK, D_MODEL), lambda i: (i, 0)),
            pl.BlockSpec((1, D_MODEL), lambda i: (0, 0)),
        ],
        out_specs=pl.BlockSpec((_ROW_BLK, D_MODEL), lambda i: (i, 0)),
        out_shape=jax.ShapeDtypeStruct((N_TOKENS, D_MODEL), jnp.float32),
    )(hidden_states, norm_weight.reshape(1, D_MODEL))


def kernel(input_ids, positions, hidden_states, embed_table, norm_weight):
    input_embeds = _sc_gather(input_ids, embed_table)
    normed = _rms_norm(hidden_states, norm_weight)
    return (normed, hidden_states, input_embeds)
